# baseline (speedup 1.0000x reference)
import functools

import jax
import jax.numpy as jnp
from jax import lax
from jax.experimental import pallas as pl
from jax.experimental.pallas import tpu as pltpu
from jax.experimental.pallas import tpu_sc as plsc

B = 4096
EMB = 16
NCAT = 1000
VIS = 512
TM = 512


def _sc_cat_prod(category, u_rows_t, i_rows_t, ct_t):
  info = plsc.get_sparse_core_info()
  nc, ns = info.num_cores, info.num_subcores
  nw = nc * ns
  bpw = B // nw

  mesh = plsc.VectorSubcoreMesh(core_axis_name="c", subcore_axis_name="s")

  @functools.partial(
      pl.kernel,
      mesh=mesh,
      out_type=jax.ShapeDtypeStruct((EMB, B), jnp.float32),
      scratch_types=[
          pltpu.VMEM((bpw,), jnp.int32),
          pltpu.VMEM((EMB, NCAT), jnp.float32),
          pltpu.VMEM((EMB, bpw), jnp.float32),
          pltpu.VMEM((EMB, bpw), jnp.float32),
          pltpu.SemaphoreType.DMA,
      ],
      compiler_params=pltpu.CompilerParams(needs_layout_passes=False),
  )
  def cat_prod_kernel(cat_hbm, u_hbm, i_hbm, ct_hbm, out_hbm, cidx, ctab, ub,
                      ib, sem):
    wid = lax.axis_index("s") * nc + lax.axis_index("c")
    base = wid * bpw
    c1 = pltpu.async_copy(cat_hbm.at[pl.ds(base, bpw)], cidx, sem)
    c2 = pltpu.async_copy(ct_hbm, ctab, sem)
    c3 = pltpu.async_copy(u_hbm.at[:, pl.ds(base, bpw)], ub, sem)
    c4 = pltpu.async_copy(i_hbm.at[:, pl.ds(base, bpw)], ib, sem)
    c1.wait()
    c2.wait()
    c3.wait()
    c4.wait()

    for g in range(bpw // 16):
      sl = pl.ds(g * 16, 16)
      cvec = cidx[sl]
      for e in range(EMB):
        ev = jnp.full((16,), e, jnp.int32)
        cv = plsc.load_gather(ctab, [ev, cvec])
        ub[e, sl] = ub[e, sl] * ib[e, sl] * cv
    pltpu.sync_copy(ub, out_hbm.at[:, pl.ds(base, bpw)])

  return cat_prod_kernel(category, u_rows_t, i_rows_t, ct_t)


def _tc_fm(scal, prod_t, visual, uf, itf, cf, Wv, bv2, Wv1):
  nb = B // TM

  def body(scal_ref, prod_ref, visual_ref, uf_ref, itf_ref, cf_ref, Wv_ref,
           bv_ref, Wv1_ref, out_ref, fo_s, pw_s):
    k = pl.program_id(0)

    @pl.when(k == 0)
    def _():
      vis = visual_ref[...]
      vemb_t = lax.dot_general(
          Wv_ref[...], vis, (((1,), (1,)), ((), ())),
          precision=lax.Precision.HIGHEST,
          preferred_element_type=jnp.float32)
      p = prod_ref[...] * (vemb_t + bv_ref[...])
      pw_s[...] = jnp.sum(p, axis=0, keepdims=True)
      vlin = lax.dot_general(
          vis, Wv1_ref[...], (((1,), (1,)), ((), ())),
          precision=lax.Precision.HIGHEST,
          preferred_element_type=jnp.float32)
      s0 = (scal_ref[1] + scal_ref[3] + scal_ref[5] + scal_ref[6] +
            scal_ref[7])
      fo_s[...] = (s0 + scal_ref[0] * uf_ref[...] +
                   scal_ref[2] * itf_ref[...] + scal_ref[4] * cf_ref[...] +
                   vlin)

    out_ref[...] = fo_s[pl.ds(k * TM, TM), :] + pw_s[...]

  return pl.pallas_call(
      body,
      grid=(nb,),
      in_specs=[
          pl.BlockSpec(memory_space=pltpu.SMEM),
          pl.BlockSpec((EMB, B), lambda k: (0, 0)),
          pl.BlockSpec((B, VIS), lambda k: (0, 0)),
          pl.BlockSpec((B, 1), lambda k: (0, 0)),
          pl.BlockSpec((B, 1), lambda k: (0, 0)),
          pl.BlockSpec((B, 1), lambda k: (0, 0)),
          pl.BlockSpec((EMB, VIS), lambda k: (0, 0)),
          pl.BlockSpec((EMB, 1), lambda k: (0, 0)),
          pl.BlockSpec((1, VIS), lambda k: (0, 0)),
      ],
      out_specs=pl.BlockSpec((TM, B), lambda k: (k, 0)),
      out_shape=jax.ShapeDtypeStruct((B, B), jnp.float32),
      scratch_shapes=[
          pltpu.VMEM((B, 1), jnp.float32),
          pltpu.VMEM((1, B), jnp.float32),
      ],
      compiler_params=pltpu.CompilerParams(
          dimension_semantics=("arbitrary",)),
  )(scal, prod_t, visual, uf, itf, cf, Wv, bv2, Wv1)


def kernel(user, item, category, visual, user_table, item_table, cat_table,
           Wv, bv, Wu, bu, Wi, bi, Wc, bc, Wv1, bv1, bias):
  u_rows_t = jnp.take(user_table, user, axis=0).T
  i_rows_t = jnp.take(item_table, item, axis=0).T
  prod_t = _sc_cat_prod(category, u_rows_t, i_rows_t, cat_table.T)
  scal = jnp.concatenate([
      Wu.reshape(-1), bu.reshape(-1), Wi.reshape(-1), bi.reshape(-1),
      Wc.reshape(-1), bc.reshape(-1), bias.reshape(-1), bv1.reshape(-1)
  ])
  uf = user.astype(jnp.float32).reshape(B, 1)
  itf = item.astype(jnp.float32).reshape(B, 1)
  cf = category.astype(jnp.float32).reshape(B, 1)
  return _tc_fm(scal, prod_t, visual, uf, itf, cf, Wv, bv.reshape(EMB, 1),
                Wv1)


# speedup vs baseline: 1.0705x; 1.0705x over previous
import functools

import jax
import jax.numpy as jnp
from jax import lax
from jax.experimental import pallas as pl
from jax.experimental.pallas import tpu as pltpu
from jax.experimental.pallas import tpu_sc as plsc

B = 4096
EMB = 16
NCAT = 1000
VIS = 512
TM = 512


def _sc_cat_gather(category, ct_t):
  info = plsc.get_sparse_core_info()
  nc, ns = info.num_cores, info.num_subcores
  nw = nc * ns
  bpw = B // nw

  mesh = plsc.VectorSubcoreMesh(core_axis_name="c", subcore_axis_name="s")

  @functools.partial(
      pl.kernel,
      mesh=mesh,
      out_type=jax.ShapeDtypeStruct((EMB, B), jnp.float32),
      scratch_types=[
          pltpu.VMEM((bpw,), jnp.int32),
          pltpu.VMEM((EMB, NCAT), jnp.float32),
          pltpu.VMEM((EMB, bpw), jnp.float32),
          pltpu.SemaphoreType.DMA,
      ],
      compiler_params=pltpu.CompilerParams(needs_layout_passes=False),
  )
  def cat_gather_kernel(cat_hbm, ct_hbm, out_hbm, cidx, ctab, cb, sem):
    wid = lax.axis_index("s") * nc + lax.axis_index("c")
    base = wid * bpw
    c1 = pltpu.async_copy(cat_hbm.at[pl.ds(base, bpw)], cidx, sem)
    c2 = pltpu.async_copy(ct_hbm, ctab, sem)
    c1.wait()
    c2.wait()

    for g in range(bpw // 16):
      sl = pl.ds(g * 16, 16)
      cvec = cidx[sl]
      for e in range(EMB):
        ev = jnp.full((16,), e, jnp.int32)
        cb[e, sl] = plsc.load_gather(ctab, [ev, cvec])
    pltpu.sync_copy(cb, out_hbm.at[:, pl.ds(base, bpw)])

  return cat_gather_kernel(category, ct_t)


def _tc_fm(scal, u_t, i_t, c_t, visual, uf, itf, cf, Wv, bv2, Wv1):
  nb = B // TM

  def body(scal_ref, u_ref, i_ref, c_ref, visual_ref, uf_ref, itf_ref,
           cf_ref, Wv_ref, bv_ref, Wv1_ref, out_ref, fo_s, pw_s):
    k = pl.program_id(0)

    @pl.when(k == 0)
    def _():
      vis = visual_ref[...]
      vemb_t = lax.dot_general(
          Wv_ref[...], vis, (((1,), (1,)), ((), ())),
          precision=lax.Precision.HIGHEST,
          preferred_element_type=jnp.float32)
      p = u_ref[...] * i_ref[...] * c_ref[...] * (vemb_t + bv_ref[...])
      pw_s[...] = jnp.sum(p, axis=0, keepdims=True)
      vlin = lax.dot_general(
          vis, Wv1_ref[...], (((1,), (1,)), ((), ())),
          precision=lax.Precision.HIGHEST,
          preferred_element_type=jnp.float32)
      s0 = (scal_ref[1] + scal_ref[3] + scal_ref[5] + scal_ref[6] +
            scal_ref[7])
      fo_s[...] = (s0 + scal_ref[0] * uf_ref[...] +
                   scal_ref[2] * itf_ref[...] + scal_ref[4] * cf_ref[...] +
                   vlin)

    out_ref[...] = fo_s[pl.ds(k * TM, TM), :] + pw_s[...]

  return pl.pallas_call(
      body,
      grid=(nb,),
      in_specs=[
          pl.BlockSpec(memory_space=pltpu.SMEM),
          pl.BlockSpec((EMB, B), lambda k: (0, 0)),
          pl.BlockSpec((EMB, B), lambda k: (0, 0)),
          pl.BlockSpec((EMB, B), lambda k: (0, 0)),
          pl.BlockSpec((B, VIS), lambda k: (0, 0)),
          pl.BlockSpec((B, 1), lambda k: (0, 0)),
          pl.BlockSpec((B, 1), lambda k: (0, 0)),
          pl.BlockSpec((B, 1), lambda k: (0, 0)),
          pl.BlockSpec((EMB, VIS), lambda k: (0, 0)),
          pl.BlockSpec((EMB, 1), lambda k: (0, 0)),
          pl.BlockSpec((1, VIS), lambda k: (0, 0)),
      ],
      out_specs=pl.BlockSpec((TM, B), lambda k: (k, 0)),
      out_shape=jax.ShapeDtypeStruct((B, B), jnp.float32),
      scratch_shapes=[
          pltpu.VMEM((B, 1), jnp.float32),
          pltpu.VMEM((1, B), jnp.float32),
      ],
      compiler_params=pltpu.CompilerParams(
          dimension_semantics=("arbitrary",)),
  )(scal, u_t, i_t, c_t, visual, uf, itf, cf, Wv, bv2, Wv1)


def kernel(user, item, category, visual, user_table, item_table, cat_table,
           Wv, bv, Wu, bu, Wi, bi, Wc, bc, Wv1, bv1, bias):
  u_rows_t = jnp.take(user_table, user, axis=0).T
  i_rows_t = jnp.take(item_table, item, axis=0).T
  c_rows_t = _sc_cat_gather(category, cat_table.T)
  scal = jnp.concatenate([
      Wu.reshape(-1), bu.reshape(-1), Wi.reshape(-1), bi.reshape(-1),
      Wc.reshape(-1), bc.reshape(-1), bias.reshape(-1), bv1.reshape(-1)
  ])
  uf = user.astype(jnp.float32).reshape(B, 1)
  itf = item.astype(jnp.float32).reshape(B, 1)
  cf = category.astype(jnp.float32).reshape(B, 1)
  return _tc_fm(scal, u_rows_t, i_rows_t, c_rows_t, visual, uf, itf, cf, Wv,
                bv.reshape(EMB, 1), Wv1)
